# baseline (speedup 1.0000x reference)
import functools

import jax
import jax.numpy as jnp
from jax import lax
from jax.experimental import pallas as pl
from jax.experimental.pallas import tpu as pltpu
from jax.experimental.pallas import tpu_sc as plsc

B = 16384
D = 64
HIDDEN = 256
NC = 2
NS = 16
NW = NC * NS
B_PER_W = B // NW
CHUNK = 128
NCHUNK = B_PER_W // CHUNK


def _gather_body(u_hbm, v_hbm, uemb_hbm, vemb_hbm, ug_hbm, vg_hbm,
                 idx_u, idx_v, rows_u, rows_v, sem):
    wid = lax.axis_index("s") * NC + lax.axis_index("c")
    base = wid * B_PER_W
    pltpu.sync_copy(u_hbm.at[pl.ds(wid * NCHUNK, NCHUNK)], idx_u)
    pltpu.sync_copy(v_hbm.at[pl.ds(wid * NCHUNK, NCHUNK)], idx_v)
    copies = []
    for j in range(NCHUNK):
        copies.append(pltpu.async_copy(
            uemb_hbm.at[idx_u.at[j]], rows_u.at[pl.ds(j * CHUNK, CHUNK)], sem))
        copies.append(pltpu.async_copy(
            vemb_hbm.at[idx_v.at[j]], rows_v.at[pl.ds(j * CHUNK, CHUNK)], sem))
    for c in copies:
        c.wait()
    pltpu.sync_copy(rows_u, ug_hbm.at[pl.ds(base, B_PER_W)])
    pltpu.sync_copy(rows_v, vg_hbm.at[pl.ds(base, B_PER_W)])


def _make_gather():
    mesh = plsc.VectorSubcoreMesh(core_axis_name="c", subcore_axis_name="s")
    return functools.partial(
        pl.kernel, mesh=mesh,
        compiler_params=pltpu.CompilerParams(use_tc_tiling_on_sc=False),
        out_type=[jax.ShapeDtypeStruct((B, D), jnp.float32),
                  jax.ShapeDtypeStruct((B, D), jnp.float32)],
        scratch_types=[
            pltpu.VMEM((NCHUNK, CHUNK), jnp.int32),
            pltpu.VMEM((NCHUNK, CHUNK), jnp.int32),
            pltpu.VMEM((B_PER_W, D), jnp.float32),
            pltpu.VMEM((B_PER_W, D), jnp.float32),
            pltpu.SemaphoreType.DMA,
        ],
    )(_gather_body)


_sc_gather = _make_gather()


def _mlp_body(xu_ref, xv_ref, w1a_ref, w1b_ref, b1_ref, w2_ref, b2_ref, o_ref):
    xu = jnp.maximum(xu_ref[...], 0.0)
    xv = jnp.maximum(xv_ref[...], 0.0)
    h = jnp.dot(xu, w1a_ref[...], preferred_element_type=jnp.float32)
    h = h + jnp.dot(xv, w1b_ref[...], preferred_element_type=jnp.float32)
    h = jnp.maximum(h + b1_ref[...], 0.0)
    o_ref[...] = jnp.dot(h, w2_ref[...], preferred_element_type=jnp.float32) + b2_ref[...]


BB = 2048


def _mlp(ug, vg, w1a, w1b, b1, w2, b2):
    grid = (B // BB,)
    return pl.pallas_call(
        _mlp_body,
        grid=grid,
        in_specs=[
            pl.BlockSpec((BB, D), lambda i: (i, 0)),
            pl.BlockSpec((BB, D), lambda i: (i, 0)),
            pl.BlockSpec((D, HIDDEN), lambda i: (0, 0)),
            pl.BlockSpec((D, HIDDEN), lambda i: (0, 0)),
            pl.BlockSpec((1, HIDDEN), lambda i: (0, 0)),
            pl.BlockSpec((HIDDEN, 1), lambda i: (0, 0)),
            pl.BlockSpec((1, 1), lambda i: (0, 0)),
        ],
        out_specs=pl.BlockSpec((BB, 1), lambda i: (i, 0)),
        out_shape=jax.ShapeDtypeStruct((B, 1), jnp.float32),
    )(ug, vg, w1a, w1b, b1, w2, b2)


def kernel(u, v, user_emb, like_emb, W1, b1, W2, b2):
    u2 = u.reshape(NW * NCHUNK, CHUNK)
    v2 = v.reshape(NW * NCHUNK, CHUNK)
    ug, vg = _sc_gather(u2, v2, user_emb, like_emb)
    return _mlp(ug, vg, W1[:D], W1[D:], b1.reshape(1, HIDDEN),
                W2, b2.reshape(1, 1))


# speedup vs baseline: 2.2370x; 2.2370x over previous
import functools

import jax
import jax.numpy as jnp
from jax import lax
from jax.experimental import pallas as pl
from jax.experimental.pallas import tpu as pltpu
from jax.experimental.pallas import tpu_sc as plsc

B = 16384
D = 64
HIDDEN = 256
NC = 2
NS = 16
NW = NC * NS
B_PER_W = B // NW
G = 4
NGRP = B_PER_W // G
LANES = 16


def _gather_one_table(idx, src, out, ring, outbuf, sems, base):
    def fire_one(j_scalar, slot, parity):
        col = pl.multiple_of((j_scalar >> 7) * 128, 128)
        pltpu.async_copy(src.at[:, pl.ds(col, 128)], ring.at[slot],
                         sems[parity])

    def fire_group(vec, lane0, parity):
        for k in range(G):
            fire_one(vec[lane0 + k], parity * G + k, parity)

    def drain(parity):
        for k in range(G):
            pltpu.make_async_copy(src.at[:, pl.ds(0, 128)],
                                  ring.at[parity * G + k], sems[parity]).wait()

    def extract_group(vec, lane0, first_elem, parity):
        for k in range(G):
            i = first_elem + k
            r = jnp.full((LANES,), vec[lane0 + k] & 127, jnp.int32)
            s = jnp.full((LANES,), parity * G + k, jnp.int32)
            for g in range(D // LANES):
                d_idx = lax.iota(jnp.int32, LANES) + (g * LANES)
                v16 = plsc.load_gather(ring, [s, d_idx, r])
                off = pl.multiple_of(i * D + g * LANES, 8)
                outbuf[pl.ds(off, LANES)] = v16

    vec0 = idx[pl.ds(0, LANES)]
    fire_group(vec0, 0, 0)
    fire_group(vec0, G, 1)

    def body(m, carry):
        off_cur = pl.multiple_of(m * LANES, 8)
        vec_cur = idx[pl.ds(off_cur, LANES)]
        for q in range(4):
            parity = q % 2
            grp = 4 * m + q
            drain(parity)
            extract_group(vec_cur, q * G, grp * G, parity)
            if q < 2:
                @pl.when(grp + 2 < NGRP)
                def _(vc=vec_cur, qq=q, par=parity):
                    fire_group(vc, (qq + 2) * G, par)
            else:
                @pl.when(grp + 2 < NGRP)
                def _(mm=m, qq=q, par=parity):
                    off_nxt = pl.multiple_of((mm + 1) * LANES, 8)
                    vec_nxt = idx[pl.ds(off_nxt, LANES)]
                    fire_group(vec_nxt, (qq - 2) * G, par)
        return carry

    lax.fori_loop(0, B_PER_W // LANES, body, 0)
    pltpu.sync_copy(outbuf, out.at[pl.ds(base * D, B_PER_W * D)])


def _gather_body(u_hbm, v_hbm, ut_hbm, vt_hbm, ug_hbm, vg_hbm,
                 idx_u, idx_v, ring, outbuf, sem_a, sem_b):
    wid = lax.axis_index("s") * NC + lax.axis_index("c")
    base = wid * B_PER_W
    pltpu.sync_copy(u_hbm.at[pl.ds(base, B_PER_W)], idx_u)
    pltpu.sync_copy(v_hbm.at[pl.ds(base, B_PER_W)], idx_v)
    sems = (sem_a, sem_b)
    _gather_one_table(idx_u, ut_hbm, ug_hbm, ring, outbuf, sems, base)
    _gather_one_table(idx_v, vt_hbm, vg_hbm, ring, outbuf, sems, base)


def _make_gather():
    mesh = plsc.VectorSubcoreMesh(core_axis_name="c", subcore_axis_name="s")
    return functools.partial(
        pl.kernel, mesh=mesh,
        compiler_params=pltpu.CompilerParams(use_tc_tiling_on_sc=True,
                                             needs_layout_passes=False),
        out_type=[jax.ShapeDtypeStruct((B * D,), jnp.float32),
                  jax.ShapeDtypeStruct((B * D,), jnp.float32)],
        scratch_types=[
            pltpu.VMEM((B_PER_W,), jnp.int32),
            pltpu.VMEM((B_PER_W,), jnp.int32),
            pltpu.VMEM((2 * G, D, 128), jnp.float32),
            pltpu.VMEM((B_PER_W * D,), jnp.float32),
            pltpu.SemaphoreType.DMA,
            pltpu.SemaphoreType.DMA,
        ],
    )(_gather_body)


_sc_gather = _make_gather()


def _mlp_body(xu_ref, xv_ref, w1a_ref, w1b_ref, b1_ref, w2_ref, b2_ref, o_ref):
    xu = jnp.maximum(xu_ref[...], 0.0)
    xv = jnp.maximum(xv_ref[...], 0.0)
    h = jnp.dot(xu, w1a_ref[...], preferred_element_type=jnp.float32)
    h = h + jnp.dot(xv, w1b_ref[...], preferred_element_type=jnp.float32)
    h = jnp.maximum(h + b1_ref[...], 0.0)
    o_ref[...] = jnp.dot(h, w2_ref[...], preferred_element_type=jnp.float32) + b2_ref[...]


BB = 2048


def _mlp(ug, vg, w1a, w1b, b1, w2, b2):
    grid = (B // BB,)
    return pl.pallas_call(
        _mlp_body,
        grid=grid,
        in_specs=[
            pl.BlockSpec((BB, D), lambda i: (i, 0)),
            pl.BlockSpec((BB, D), lambda i: (i, 0)),
            pl.BlockSpec((D, HIDDEN), lambda i: (0, 0)),
            pl.BlockSpec((D, HIDDEN), lambda i: (0, 0)),
            pl.BlockSpec((1, HIDDEN), lambda i: (0, 0)),
            pl.BlockSpec((HIDDEN, 1), lambda i: (0, 0)),
            pl.BlockSpec((1, 1), lambda i: (0, 0)),
        ],
        out_specs=pl.BlockSpec((BB, 1), lambda i: (i, 0)),
        out_shape=jax.ShapeDtypeStruct((B, 1), jnp.float32),
    )(ug, vg, w1a, w1b, b1, w2, b2)


def kernel(u, v, user_emb, like_emb, W1, b1, W2, b2):
    ug1, vg1 = _sc_gather(u, v, user_emb.T, like_emb.T)
    ug = ug1.reshape(B, D)
    vg = vg1.reshape(B, D)
    return _mlp(ug, vg, W1[:D], W1[D:], b1.reshape(1, HIDDEN),
                W2, b2.reshape(1, 1))
